# baseline (speedup 1.0000x reference)
import jax
import jax.numpy as jnp
from jax.experimental import pallas as pl
from jax.experimental.pallas import tpu as pltpu

_G = 256
_EPS = 1e-6
_BLK = 1024


def _onehot16(b, n):
    ids = jax.lax.broadcasted_iota(jnp.int32, (b.shape[0], n), 1)
    return (b[:, None] == ids).astype(jnp.bfloat16)


def _stats_kernel(xb_ref, bb_ref, s_ref):
    i = pl.program_id(0)
    xb = xb_ref[...]
    oh = _onehot16(bb_ref[0, 0, :], _G)
    x16 = xb.astype(jnp.bfloat16)
    xsq16 = (xb * xb).astype(jnp.bfloat16)
    ones = jnp.ones((xb.shape[0], 128), jnp.bfloat16)
    lhs = jnp.concatenate([x16, xsq16, ones], axis=1)
    dims = (((0,), (0,)), ((), ()))
    s = jax.lax.dot_general(oh, lhs, dims, preferred_element_type=jnp.float32)

    @pl.when(i == 0)
    def _():
        s_ref[...] = jnp.zeros_like(s_ref)

    s_ref[...] += s


def _norm_kernel(s_ref, w_ref, bias_ref, ms_ref, xb_ref, bb_ref, y_ref,
                 tab_ref):
    i = pl.program_id(0)
    d = xb_ref.shape[1]

    @pl.when(i == 0)
    def _():
        inv_c = 1.0 / jnp.maximum(s_ref[:, 2 * d:2 * d + 1], 1.0)
        m = s_ref[:, :d] * inv_c
        ex2 = s_ref[:, d:2 * d] * inv_c
        s = ms_ref[...]
        var = ex2 - m * m * (s * (2.0 - s))
        inv_std = jax.lax.rsqrt(var + _EPS)
        a = w_ref[...] * inv_std
        b = bias_ref[...] - a * m * s
        a_hi = a.astype(jnp.bfloat16)
        a_lo = (a - a_hi.astype(jnp.float32)).astype(jnp.bfloat16)
        tab_ref[...] = jnp.concatenate(
            [a_hi, a_lo, b.astype(jnp.bfloat16)], axis=1)

    xb = xb_ref[...]
    oh = _onehot16(bb_ref[0, 0, :], _G)
    dims = (((1,), (0,)), ((), ()))
    rows = jax.lax.dot_general(oh, tab_ref[...], dims,
                               preferred_element_type=jnp.float32)
    y_ref[...] = (rows[:, :d] + rows[:, d:2 * d]) * xb + rows[:, 2 * d:]


@jax.jit
def kernel(x, batch, weight, bias, mean_scale):
    n, d = x.shape
    batch = batch.astype(jnp.int32)
    nb = (n + _BLK - 1) // _BLK
    npad = nb * _BLK
    x_p = jnp.pad(x, ((0, npad - n), (0, 0)))
    b_p = jnp.pad(batch, (0, npad - n), constant_values=_G)
    b_p = b_p.reshape(nb, 1, _BLK)

    full = lambda i: (0, 0)
    stats = pl.pallas_call(
        _stats_kernel,
        grid=(nb,),
        in_specs=[
            pl.BlockSpec((_BLK, d), lambda i: (i, 0)),
            pl.BlockSpec((1, 1, _BLK), lambda i: (i, 0, 0)),
        ],
        out_specs=pl.BlockSpec((_G, 2 * d + 128), full),
        out_shape=jax.ShapeDtypeStruct((_G, 2 * d + 128), jnp.float32),
    )(x_p, b_p)

    w2 = weight.reshape(1, d)
    bi2 = bias.reshape(1, d)
    ms2 = mean_scale.reshape(1, d)
    y = pl.pallas_call(
        _norm_kernel,
        grid=(nb,),
        in_specs=[
            pl.BlockSpec((_G, 2 * d + 128), full),
            pl.BlockSpec((1, d), full),
            pl.BlockSpec((1, d), full),
            pl.BlockSpec((1, d), full),
            pl.BlockSpec((_BLK, d), lambda i: (i, 0)),
            pl.BlockSpec((1, 1, _BLK), lambda i: (i, 0, 0)),
        ],
        out_specs=pl.BlockSpec((_BLK, d), lambda i: (i, 0)),
        out_shape=jax.ShapeDtypeStruct((npad, d), jnp.float32),
        scratch_shapes=[
            pltpu.VMEM((_G, 3 * d), jnp.bfloat16),
        ],
    )(stats, w2, bi2, ms2, x_p, b_p)
    return y[:n]
